# Initial kernel scaffold; baseline (speedup 1.0000x reference)
import jax
import jax.numpy as jnp
from jax.experimental import pallas as pl


def kernel(categorical_feats, tables):
    raise NotImplementedError("write your pallas kernel here")


# speedup vs baseline: 1.1692x; 1.1692x over previous
import functools

import jax
import jax.numpy as jnp
from jax import lax
from jax.experimental import pallas as pl
from jax.experimental.pallas import tpu as pltpu
from jax.experimental.pallas import tpu_sc as plsc

_NUM_FIELDS = 26
_VOCAB = 100000
_EMB = 32
_BATCH = 16384

_NC = 2
_NS = 16
_NW = _NC * _NS
_BPW = _BATCH // _NW
_CHUNK = 128
_NQ = _BPW // _CHUNK
_L = 16


def _body(feats_hbm, table_hbm, out_hbm, feats_v, idx_v, rows_v, acc_v,
          sem0, sem1):
    wid = lax.axis_index("s") * _NC + lax.axis_index("c")
    base = wid * _BPW
    pltpu.sync_copy(feats_hbm.at[:, pl.ds(base, _BPW)], feats_v)

    sems = (sem0, sem1)
    n_chunks = _NUM_FIELDS * _NQ

    def build_idx(c, f, q):
        buf = c % 2

        def jbody(j, carry):
            off = q * _CHUNK + j * _L
            vals = feats_v[f, pl.ds(off, _L)] + f * _VOCAB
            idx_v[buf, pl.ds(j * _L, _L)] = vals
            return carry

        lax.fori_loop(0, _CHUNK // _L, jbody, 0, unroll=2)

    def accumulate(c, f, q):
        buf = c % 2
        first = f == 0

        def ibody(i, carry):
            r = i * 4
            for rr in range(4):
                row = q * _CHUNK + r + rr
                for h in range(2):
                    v = rows_v[buf, r + rr, pl.ds(h * _L, _L)]
                    if first:
                        acc_v[row, pl.ds(h * _L, _L)] = v
                    else:
                        plsc.addupdate(acc_v.at[row, pl.ds(h * _L, _L)], v)
            return carry

        lax.fori_loop(0, _CHUNK // 4, ibody, 0)

    cps = [None, None]
    for c in range(n_chunks):
        f, q = divmod(c, _NQ)
        build_idx(c, f, q)
        cps[c % 2] = pltpu.async_copy(
            table_hbm.at[idx_v.at[c % 2]], rows_v.at[c % 2], sems[c % 2])
        if c > 0:
            cps[(c - 1) % 2].wait()
            accumulate(c - 1, *divmod(c - 1, _NQ))
    cps[(n_chunks - 1) % 2].wait()
    accumulate(n_chunks - 1, *divmod(n_chunks - 1, _NQ))

    pltpu.sync_copy(acc_v, out_hbm.at[pl.ds(base, _BPW)])


_embed_sum = functools.partial(
    pl.kernel,
    out_type=jax.ShapeDtypeStruct((_BATCH, _EMB), jnp.float32),
    mesh=plsc.VectorSubcoreMesh(core_axis_name="c", subcore_axis_name="s"),
    compiler_params=pltpu.CompilerParams(use_tc_tiling_on_sc=False),
    scratch_types=[
        pltpu.VMEM((_NUM_FIELDS, _BPW), jnp.int32),
        pltpu.VMEM((2, _CHUNK), jnp.int32),
        pltpu.VMEM((2, _CHUNK, _EMB), jnp.float32),
        pltpu.VMEM((_BPW, _EMB), jnp.float32),
        pltpu.SemaphoreType.DMA,
        pltpu.SemaphoreType.DMA,
    ],
)(_body)


def kernel(categorical_feats, tables):
    feats_t = categorical_feats.astype(jnp.int32).T
    table = tables.reshape(_NUM_FIELDS * _VOCAB, _EMB)
    return _embed_sum(feats_t, table)
